# baseline (speedup 1.0000x reference)
import functools

import jax
import jax.numpy as jnp
import numpy as np
from jax import lax
from jax.experimental import pallas as pl
from jax.experimental.pallas import tpu as pltpu
from jax.experimental.pallas import tpu_sc as plsc

_BS = 2
_NQ = 5440
_NV = 5440
_D = 256
_NH = 8
_NL = 4
_NP = 4
_HD = 32
_SPATIAL = [(64, 64), (32, 32), (16, 16), (8, 8)]
_SIZES = [h * w for (h, w) in _SPATIAL]
_STARTS = np.concatenate([[0], np.cumsum(_SIZES)]).astype(np.int32)
_NK = _NL * _NP * 4
_QB = 680
_QC = 160

_PITCH = 17
_lidx = np.tile(np.repeat(np.arange(_NL), _NP), _NH)
_Wc = np.array([_SPATIAL[l][1] for l in _lidx], np.float32)
_Hc = np.array([_SPATIAL[l][0] for l in _lidx], np.float32)
_S16c = np.array([_STARTS[l] * _PITCH for l in _lidx], np.float32)

_cols = np.arange(_NH * _NL * _NP * 2).reshape(_NH, _NL, _NP, 2)
_PERM = np.concatenate([_cols[..., 0].reshape(-1), _cols[..., 1].reshape(-1)])

_BD = np.kron(np.eye(_NH, dtype=np.float32), np.ones((16, 16), np.float32))

_CONSTS = np.zeros((8, 128), np.float32)
_CONSTS[0] = _Wc
_CONSTS[1] = _Hc
_CONSTS[2] = _S16c


def _prep_kernel(q_ref, rpx_ref, rpy_ref, woffx_ref, woffy_ref, boffx_ref,
                 boffy_ref, wattn_ref, battn_ref, bd_ref, consts_ref,
                 idx0_ref, idx1_ref, idx2_ref, idx3_ref,
                 w0_ref, w1_ref, w2_ref, w3_ref):
    q = q_ref[0]
    offx = jnp.dot(q, woffx_ref[...], preferred_element_type=jnp.float32) + boffx_ref[...]
    offy = jnp.dot(q, woffy_ref[...], preferred_element_type=jnp.float32) + boffy_ref[...]
    awr = jnp.dot(q, wattn_ref[...], preferred_element_type=jnp.float32) + battn_ref[...]
    e = jnp.exp(awr)
    s = jnp.dot(e, bd_ref[...], preferred_element_type=jnp.float32)
    aw = e / s

    wv = consts_ref[0:1, :]
    hv = consts_ref[1:2, :]
    s16 = consts_ref[2:3, :]
    gx = 2.0 * (rpx_ref[0] + offx * (1.0 / wv)) - 1.0
    gy = 2.0 * (rpy_ref[0] + offy * (1.0 / hv)) - 1.0
    x = ((gx + 1.0) * wv - 1.0) * 0.5
    y = ((gy + 1.0) * hv - 1.0) * 0.5

    x0 = jnp.floor(x)
    y0 = jnp.floor(y)
    lx = x - x0
    ly = y - y0
    hx = 1.0 - lx
    hy = 1.0 - ly

    outs = ((idx0_ref, w0_ref, x0, y0, hx * hy),
            (idx1_ref, w1_ref, x0 + 1.0, y0, lx * hy),
            (idx2_ref, w2_ref, x0, y0 + 1.0, hx * ly),
            (idx3_ref, w3_ref, x0 + 1.0, y0 + 1.0, lx * ly))
    for idx_ref, w_ref, xi, yi, bw in outs:
        valid = ((xi >= 0.0) & (xi <= wv - 1.0)
                 & (yi >= 0.0) & (yi <= hv - 1.0)).astype(jnp.float32)
        xc = jnp.clip(xi, 0.0, wv - 1.0)
        yc = jnp.clip(yi, 0.0, hv - 1.0)
        idx16 = (yc * wv + xc) * float(_PITCH) + s16
        idx_ref[0] = idx16.astype(jnp.int32)
        w_ref[0] = bw * valid * aw


def _prep(query, rpx, rpy, woffx, woffy, boffx, boffy, wattn, battn):
    n128 = jax.ShapeDtypeStruct((_BS, _NQ, 128), jnp.float32)
    i128 = jax.ShapeDtypeStruct((_BS, _NQ, 128), jnp.int32)
    blk = lambda: pl.BlockSpec((1, _QB, 128), lambda b, i: (b, i, 0))
    return pl.pallas_call(
        _prep_kernel,
        grid=(_BS, _NQ // _QB),
        in_specs=[
            pl.BlockSpec((1, _QB, _D), lambda b, i: (b, i, 0)),
            blk(), blk(),
            pl.BlockSpec((_D, 128), lambda b, i: (0, 0)),
            pl.BlockSpec((_D, 128), lambda b, i: (0, 0)),
            pl.BlockSpec((128,), lambda b, i: (0,)),
            pl.BlockSpec((128,), lambda b, i: (0,)),
            pl.BlockSpec((_D, 128), lambda b, i: (0, 0)),
            pl.BlockSpec((128,), lambda b, i: (0,)),
            pl.BlockSpec((128, 128), lambda b, i: (0, 0)),
            pl.BlockSpec((8, 128), lambda b, i: (0, 0)),
        ],
        out_specs=[blk()] * 8,
        out_shape=[i128, i128, i128, i128, n128, n128, n128, n128],
    )(query, rpx, rpy, woffx, woffy, boffx, boffy, wattn, battn,
      jnp.asarray(_BD), jnp.asarray(_CONSTS))


def _proj_kernel(x_ref, w_ref, b_ref, o_ref):
    o_ref[...] = jnp.dot(x_ref[...], w_ref[...],
                         preferred_element_type=jnp.float32) + b_ref[...]


def _proj(x2d, W, b, blk=_QB):
    n, d = x2d.shape
    dout = W.shape[1]
    return pl.pallas_call(
        _proj_kernel,
        grid=(n // blk,),
        in_specs=[
            pl.BlockSpec((blk, d), lambda i: (i, 0)),
            pl.BlockSpec((d, dout), lambda i: (0, 0)),
            pl.BlockSpec((dout,), lambda i: (0,)),
        ],
        out_specs=pl.BlockSpec((blk, dout), lambda i: (i, 0)),
        out_shape=jax.ShapeDtypeStruct((n, dout), jnp.float32),
    )(x2d, W, b)


_NCH = _NQ // _QC
_CW = _NK * _QC
_OW = _QC * 16
_VTW = _NV * _PITCH
_KU = 8


def _sc_gather_body(vt_hbm, idx_hbm, w_hbm, out_hbm, vt_v, idx_v, w_v, out_v):
    cid = lax.axis_index("c")
    sid = lax.axis_index("s")
    wid = sid * 2 + cid
    bh = wid // 2
    pltpu.sync_copy(vt_hbm.at[pl.ds(wid * _VTW, _VTW)], vt_v)

    def chunk_body(ci, _):
        iw_base = (bh * _NCH + ci) * _CW
        pltpu.sync_copy(idx_hbm.at[pl.ds(iw_base, _CW)], idx_v)
        pltpu.sync_copy(w_hbm.at[pl.ds(iw_base, _CW)], w_v)

        def group_body(g, _):
            ql = g * 16
            def k_body(ko, accs):
                for ku in range(_KU):
                    kq = (ko * _KU + ku) * _QC + ql
                    idxv = idx_v[pl.ds(kq, 16)]
                    wvec = w_v[pl.ds(kq, 16)]
                    accs = tuple(
                        accs[c] + wvec * plsc.load_gather(vt_v, [idxv + c])
                        for c in range(16))
                return accs
            accs = lax.fori_loop(
                0, _NK // _KU, k_body,
                tuple(jnp.zeros((16,), jnp.float32) for _ in range(16)))
            rows = (lax.iota(jnp.int32, 16) + ql) * 16
            for c in range(16):
                plsc.store_scatter(out_v, [rows + c], accs[c])
            return 0

        lax.fori_loop(0, _QC // 16, group_body, 0)
        pltpu.sync_copy(out_v, out_hbm.at[pl.ds((wid * _NCH + ci) * _OW, _OW)])
        return 0

    lax.fori_loop(0, _NCH, chunk_body, 0)


@functools.cache
def _sc_gather_build():
    return pl.kernel(
        _sc_gather_body,
        out_type=jax.ShapeDtypeStruct((_BS * _NH * 2 * _NQ * 16,), jnp.float32),
        mesh=plsc.VectorSubcoreMesh(core_axis_name="c", subcore_axis_name="s",
                                    num_cores=2, num_subcores=16),
        compiler_params=pltpu.CompilerParams(needs_layout_passes=False),
        scratch_types=[
            pltpu.VMEM((_VTW,), jnp.float32),
            pltpu.VMEM((_CW,), jnp.int32),
            pltpu.VMEM((_CW,), jnp.float32),
            pltpu.VMEM((_OW,), jnp.float32),
        ],
    )


def _sc_gather(vt, idx_sc, w_sc):
    return _sc_gather_build()(vt, idx_sc, w_sc)


def kernel(query, value, reference_points, W_value, b_value, W_off, b_off,
           W_attn, b_attn, W_out, b_out):
    woffp = W_off[:, _PERM]
    boffp = b_off[_PERM]
    woffx, woffy = woffp[:, :128], woffp[:, 128:]
    boffx, boffy = boffp[:128], boffp[128:]
    rp = jnp.broadcast_to(reference_points[:, :, None, :, None, :],
                          (_BS, _NQ, _NH, _NL, _NP, 2))
    rpx = rp[..., 0].reshape(_BS, _NQ, 128)
    rpy = rp[..., 1].reshape(_BS, _NQ, 128)

    i0, i1, i2, i3, w0, w1, w2, w3 = _prep(
        query, rpx, rpy, woffx, woffy, boffx, boffy, W_attn, b_attn)

    def to_sc(parts):
        a = jnp.stack(parts, axis=2)
        a = a.reshape(_BS, _NCH, _QC, 4, _NH, 16)
        return jnp.transpose(a, (0, 4, 1, 3, 5, 2)).reshape(-1)

    idx_sc = to_sc([i0, i1, i2, i3])
    w_sc = to_sc([w0, w1, w2, w3])

    v = _proj(value.reshape(_BS * _NV, _D), W_value, b_value)
    vt = jnp.transpose(v.reshape(_BS, _NV, _NH, 2, 16),
                       (0, 2, 3, 1, 4))
    vt = jnp.pad(vt, ((0, 0), (0, 0), (0, 0), (0, 0),
                      (0, _PITCH - 16))).reshape(-1)

    out_sc = _sc_gather(vt, idx_sc, w_sc)

    out2d = jnp.transpose(out_sc.reshape(_BS, _NH, 2, _NCH, _QC, 16),
                          (0, 3, 4, 1, 2, 5)).reshape(_BS * _NQ, _D)
    return _proj(out2d, W_out, b_out).reshape(_BS, _NQ, _D)


# speedup vs baseline: 1.6625x; 1.6625x over previous
import functools

import jax
import jax.numpy as jnp
import numpy as np
from jax import lax
from jax.experimental import pallas as pl
from jax.experimental.pallas import tpu as pltpu
from jax.experimental.pallas import tpu_sc as plsc

_BS = 2
_NQ = 5440
_NV = 5440
_D = 256
_NH = 8
_NL = 4
_NP = 4
_HD = 32
_SPATIAL = [(64, 64), (32, 32), (16, 16), (8, 8)]
_SIZES = [h * w for (h, w) in _SPATIAL]
_STARTS = np.concatenate([[0], np.cumsum(_SIZES)]).astype(np.int32)
_NK = _NL * _NP * 4
_QB = 680
_QC = 160

_PITCH = 17
_lidx = np.tile(np.repeat(np.arange(_NL), _NP), _NH)
_Wc = np.array([_SPATIAL[l][1] for l in _lidx], np.float32)
_Hc = np.array([_SPATIAL[l][0] for l in _lidx], np.float32)
_S16c = np.array([_STARTS[l] * _PITCH for l in _lidx], np.float32)

_cols = np.arange(_NH * _NL * _NP * 2).reshape(_NH, _NL, _NP, 2)
_PERM = np.concatenate([_cols[..., 0].reshape(-1), _cols[..., 1].reshape(-1)])

_BD = np.kron(np.eye(_NH, dtype=np.float32), np.ones((16, 16), np.float32))

_CONSTS = np.zeros((8, 128), np.float32)
_CONSTS[0] = _Wc
_CONSTS[1] = _Hc
_CONSTS[2] = _S16c


def _prep_kernel(q_ref, rpx_ref, rpy_ref, woffx_ref, woffy_ref, boffx_ref,
                 boffy_ref, wattn_ref, battn_ref, bd_ref, consts_ref,
                 idx0_ref, idx1_ref, idx2_ref, idx3_ref,
                 w0_ref, w1_ref, w2_ref, w3_ref):
    q = q_ref[0]
    offx = jnp.dot(q, woffx_ref[...], preferred_element_type=jnp.float32) + boffx_ref[...]
    offy = jnp.dot(q, woffy_ref[...], preferred_element_type=jnp.float32) + boffy_ref[...]
    awr = jnp.dot(q, wattn_ref[...], preferred_element_type=jnp.float32) + battn_ref[...]
    e = jnp.exp(awr)
    s = jnp.dot(e, bd_ref[...], preferred_element_type=jnp.float32)
    aw = e / s

    wv = consts_ref[0:1, :]
    hv = consts_ref[1:2, :]
    s16 = consts_ref[2:3, :]
    gx = 2.0 * (rpx_ref[0] + offx * (1.0 / wv)) - 1.0
    gy = 2.0 * (rpy_ref[0] + offy * (1.0 / hv)) - 1.0
    x = ((gx + 1.0) * wv - 1.0) * 0.5
    y = ((gy + 1.0) * hv - 1.0) * 0.5

    x0 = jnp.floor(x)
    y0 = jnp.floor(y)
    lx = x - x0
    ly = y - y0
    hx = 1.0 - lx
    hy = 1.0 - ly

    outs = ((idx0_ref, w0_ref, x0, y0, hx * hy),
            (idx1_ref, w1_ref, x0 + 1.0, y0, lx * hy),
            (idx2_ref, w2_ref, x0, y0 + 1.0, hx * ly),
            (idx3_ref, w3_ref, x0 + 1.0, y0 + 1.0, lx * ly))
    for idx_ref, w_ref, xi, yi, bw in outs:
        valid = ((xi >= 0.0) & (xi <= wv - 1.0)
                 & (yi >= 0.0) & (yi <= hv - 1.0)).astype(jnp.float32)
        xc = jnp.clip(xi, 0.0, wv - 1.0)
        yc = jnp.clip(yi, 0.0, hv - 1.0)
        idx16 = (yc * wv + xc) * float(_PITCH) + s16
        idx_ref[0] = idx16.astype(jnp.int32)
        w_ref[0] = bw * valid * aw


def _prep(query, rpx, rpy, woffx, woffy, boffx, boffy, wattn, battn):
    n128 = jax.ShapeDtypeStruct((_BS, _NQ, 128), jnp.float32)
    i128 = jax.ShapeDtypeStruct((_BS, _NQ, 128), jnp.int32)
    blk = lambda: pl.BlockSpec((1, _QB, 128), lambda b, i: (b, i, 0))
    return pl.pallas_call(
        _prep_kernel,
        grid=(_BS, _NQ // _QB),
        in_specs=[
            pl.BlockSpec((1, _QB, _D), lambda b, i: (b, i, 0)),
            blk(), blk(),
            pl.BlockSpec((_D, 128), lambda b, i: (0, 0)),
            pl.BlockSpec((_D, 128), lambda b, i: (0, 0)),
            pl.BlockSpec((128,), lambda b, i: (0,)),
            pl.BlockSpec((128,), lambda b, i: (0,)),
            pl.BlockSpec((_D, 128), lambda b, i: (0, 0)),
            pl.BlockSpec((128,), lambda b, i: (0,)),
            pl.BlockSpec((128, 128), lambda b, i: (0, 0)),
            pl.BlockSpec((8, 128), lambda b, i: (0, 0)),
        ],
        out_specs=[blk()] * 8,
        out_shape=[i128, i128, i128, i128, n128, n128, n128, n128],
    )(query, rpx, rpy, woffx, woffy, boffx, boffy, wattn, battn,
      jnp.asarray(_BD), jnp.asarray(_CONSTS))


def _proj_kernel(x_ref, w_ref, b_ref, o_ref):
    o_ref[...] = jnp.dot(x_ref[...], w_ref[...],
                         preferred_element_type=jnp.float32) + b_ref[...]


def _proj(x2d, W, b, blk=_QB):
    n, d = x2d.shape
    dout = W.shape[1]
    return pl.pallas_call(
        _proj_kernel,
        grid=(n // blk,),
        in_specs=[
            pl.BlockSpec((blk, d), lambda i: (i, 0)),
            pl.BlockSpec((d, dout), lambda i: (0, 0)),
            pl.BlockSpec((dout,), lambda i: (0,)),
        ],
        out_specs=pl.BlockSpec((blk, dout), lambda i: (i, 0)),
        out_shape=jax.ShapeDtypeStruct((n, dout), jnp.float32),
    )(x2d, W, b)


_NCH = _NQ // _QC
_CW = _NK * _QC
_OW = _QC * 16
_VTW = _NV * _PITCH
_KU = 1


def _sc_gather_body(vt_hbm, idx_hbm, w_hbm, out_hbm, vt_v, idx_v, w_v, out_v):
    cid = lax.axis_index("c")
    sid = lax.axis_index("s")
    wid = sid * 2 + cid
    bh = wid // 2
    pltpu.sync_copy(vt_hbm.at[pl.ds(wid * _VTW, _VTW)], vt_v)

    def chunk_body(ci, _):
        iw_base = (bh * _NCH + ci) * _CW
        pltpu.sync_copy(idx_hbm.at[pl.ds(iw_base, _CW)], idx_v)
        pltpu.sync_copy(w_hbm.at[pl.ds(iw_base, _CW)], w_v)

        def group_body(g, _):
            ql = g * 16
            def k_body(ko, accs):
                for ku in range(_KU):
                    kq = (ko * _KU + ku) * _QC + ql
                    idxv = idx_v[pl.ds(kq, 16)]
                    wvec = w_v[pl.ds(kq, 16)]
                    accs = tuple(
                        accs[c] + wvec * plsc.load_gather(vt_v, [idxv + c])
                        for c in range(16))
                return accs
            accs = lax.fori_loop(
                0, _NK // _KU, k_body,
                tuple(jnp.zeros((16,), jnp.float32) for _ in range(16)))
            rows = (lax.iota(jnp.int32, 16) + ql) * 16
            for c in range(16):
                plsc.store_scatter(out_v, [rows + c], accs[c])
            return 0

        lax.fori_loop(0, _QC // 16, group_body, 0)
        pltpu.sync_copy(out_v, out_hbm.at[pl.ds((wid * _NCH + ci) * _OW, _OW)])
        return 0

    lax.fori_loop(0, _NCH, chunk_body, 0)


@functools.cache
def _sc_gather_build():
    return pl.kernel(
        _sc_gather_body,
        out_type=jax.ShapeDtypeStruct((_BS * _NH * 2 * _NQ * 16,), jnp.float32),
        mesh=plsc.VectorSubcoreMesh(core_axis_name="c", subcore_axis_name="s",
                                    num_cores=2, num_subcores=16),
        compiler_params=pltpu.CompilerParams(needs_layout_passes=False),
        scratch_types=[
            pltpu.VMEM((_VTW,), jnp.float32),
            pltpu.VMEM((_CW,), jnp.int32),
            pltpu.VMEM((_CW,), jnp.float32),
            pltpu.VMEM((_OW,), jnp.float32),
        ],
    )


def _sc_gather(vt, idx_sc, w_sc):
    return _sc_gather_build()(vt, idx_sc, w_sc)


def kernel(query, value, reference_points, W_value, b_value, W_off, b_off,
           W_attn, b_attn, W_out, b_out):
    woffp = W_off[:, _PERM]
    boffp = b_off[_PERM]
    woffx, woffy = woffp[:, :128], woffp[:, 128:]
    boffx, boffy = boffp[:128], boffp[128:]
    rp = jnp.broadcast_to(reference_points[:, :, None, :, None, :],
                          (_BS, _NQ, _NH, _NL, _NP, 2))
    rpx = rp[..., 0].reshape(_BS, _NQ, 128)
    rpy = rp[..., 1].reshape(_BS, _NQ, 128)

    i0, i1, i2, i3, w0, w1, w2, w3 = _prep(
        query, rpx, rpy, woffx, woffy, boffx, boffy, W_attn, b_attn)

    def to_sc(parts):
        a = jnp.stack(parts, axis=2)
        a = a.reshape(_BS, _NCH, _QC, 4, _NH, 16)
        return jnp.transpose(a, (0, 4, 1, 3, 5, 2)).reshape(-1)

    idx_sc = to_sc([i0, i1, i2, i3])
    w_sc = to_sc([w0, w1, w2, w3])

    v = _proj(value.reshape(_BS * _NV, _D), W_value, b_value)
    vt = jnp.transpose(v.reshape(_BS, _NV, _NH, 2, 16),
                       (0, 2, 3, 1, 4))
    vt = jnp.pad(vt, ((0, 0), (0, 0), (0, 0), (0, 0),
                      (0, _PITCH - 16))).reshape(-1)

    out_sc = _sc_gather(vt, idx_sc, w_sc)

    out2d = jnp.transpose(out_sc.reshape(_BS, _NH, 2, _NCH, _QC, 16),
                          (0, 3, 4, 1, 2, 5)).reshape(_BS * _NQ, _D)
    return _proj(out2d, W_out, b_out).reshape(_BS, _NQ, _D)


# speedup vs baseline: 2.1009x; 1.2637x over previous
import functools

import jax
import jax.numpy as jnp
import numpy as np
from jax import lax
from jax.experimental import pallas as pl
from jax.experimental.pallas import tpu as pltpu
from jax.experimental.pallas import tpu_sc as plsc

_BS = 2
_NQ = 5440
_NV = 5440
_D = 256
_NH = 8
_NL = 4
_NP = 4
_HD = 32
_SPATIAL = [(64, 64), (32, 32), (16, 16), (8, 8)]
_SIZES = [h * w for (h, w) in _SPATIAL]
_STARTS = np.concatenate([[0], np.cumsum(_SIZES)]).astype(np.int32)
_NK = _NL * _NP * 4
_QB = 680
_QC = 160

_PITCH = 17
_lidx = np.tile(np.repeat(np.arange(_NL), _NP), _NH)
_Wc = np.array([_SPATIAL[l][1] for l in _lidx], np.float32)
_Hc = np.array([_SPATIAL[l][0] for l in _lidx], np.float32)
_S16c = np.array([_STARTS[l] * _PITCH for l in _lidx], np.float32)

_cols = np.arange(_NH * _NL * _NP * 2).reshape(_NH, _NL, _NP, 2)
_PERM = np.concatenate([_cols[..., 0].reshape(-1), _cols[..., 1].reshape(-1)])

_BD = np.kron(np.eye(_NH, dtype=np.float32), np.ones((16, 16), np.float32))

_CONSTS = np.zeros((8, 128), np.float32)
_CONSTS[0] = _Wc
_CONSTS[1] = _Hc
_CONSTS[2] = _S16c


def _prep_kernel(q_ref, rpx_ref, rpy_ref, woffx_ref, woffy_ref, boffx_ref,
                 boffy_ref, wattn_ref, battn_ref, bd_ref, consts_ref,
                 idx_ref, w_ref):
    q = q_ref[0]
    offx = jnp.dot(q, woffx_ref[...], preferred_element_type=jnp.float32) + boffx_ref[...]
    offy = jnp.dot(q, woffy_ref[...], preferred_element_type=jnp.float32) + boffy_ref[...]
    awr = jnp.dot(q, wattn_ref[...], preferred_element_type=jnp.float32) + battn_ref[...]
    e = jnp.exp(awr)
    s = jnp.dot(e, bd_ref[...], preferred_element_type=jnp.float32)
    aw = e / s

    wv = consts_ref[0:1, :]
    hv = consts_ref[1:2, :]
    s16 = consts_ref[2:3, :]
    gx = 2.0 * (rpx_ref[0] + offx * (1.0 / wv)) - 1.0
    gy = 2.0 * (rpy_ref[0] + offy * (1.0 / hv)) - 1.0
    x = ((gx + 1.0) * wv - 1.0) * 0.5
    y = ((gy + 1.0) * hv - 1.0) * 0.5

    x0 = jnp.floor(x)
    y0 = jnp.floor(y)
    lx = x - x0
    ly = y - y0
    hx = 1.0 - lx
    hy = 1.0 - ly

    corners = ((x0, y0, hx * hy),
               (x0 + 1.0, y0, lx * hy),
               (x0, y0 + 1.0, hx * ly),
               (x0 + 1.0, y0 + 1.0, lx * ly))
    idx_t, w_t = [], []
    for xi, yi, bw in corners:
        valid = ((xi >= 0.0) & (xi <= wv - 1.0)
                 & (yi >= 0.0) & (yi <= hv - 1.0)).astype(jnp.float32)
        xc = jnp.clip(xi, 0.0, wv - 1.0)
        yc = jnp.clip(yi, 0.0, hv - 1.0)
        idx16 = (yc * wv + xc) * float(_PITCH) + s16
        idx_t.append(jnp.transpose(idx16, (1, 0)))
        w_t.append(jnp.transpose(bw * valid * aw, (1, 0)))
    idx_rows = jnp.concatenate(
        [t[h * 16:(h + 1) * 16] for h in range(_NH) for t in idx_t], axis=0)
    w_rows = jnp.concatenate(
        [t[h * 16:(h + 1) * 16] for h in range(_NH) for t in w_t], axis=0)
    idx_ref[0, 0] = idx_rows
    w_ref[0, 0] = w_rows


def _prep(query, rpx, rpy, woffx, woffy, boffx, boffy, wattn, battn):
    sc_arr = jax.ShapeDtypeStruct((_BS, _NQ // _QC, 512, _QC), jnp.float32)
    blk = lambda: pl.BlockSpec((1, _QC, 128), lambda b, i: (b, i, 0))
    return pl.pallas_call(
        _prep_kernel,
        grid=(_BS, _NQ // _QC),
        in_specs=[
            pl.BlockSpec((1, _QC, _D), lambda b, i: (b, i, 0)),
            blk(), blk(),
            pl.BlockSpec((_D, 128), lambda b, i: (0, 0)),
            pl.BlockSpec((_D, 128), lambda b, i: (0, 0)),
            pl.BlockSpec((128,), lambda b, i: (0,)),
            pl.BlockSpec((128,), lambda b, i: (0,)),
            pl.BlockSpec((_D, 128), lambda b, i: (0, 0)),
            pl.BlockSpec((128,), lambda b, i: (0,)),
            pl.BlockSpec((128, 128), lambda b, i: (0, 0)),
            pl.BlockSpec((8, 128), lambda b, i: (0, 0)),
        ],
        out_specs=[pl.BlockSpec((1, 1, 512, _QC), lambda b, i: (b, i, 0, 0))] * 2,
        out_shape=[sc_arr, sc_arr],
    )(query, rpx, rpy, woffx, woffy, boffx, boffy, wattn, battn,
      jnp.asarray(_BD), jnp.asarray(_CONSTS))


def _proj_kernel(x_ref, w_ref, b_ref, o_ref):
    o_ref[...] = jnp.dot(x_ref[...], w_ref[...],
                         preferred_element_type=jnp.float32) + b_ref[...]


def _proj(x2d, W, b, blk=_QB):
    n, d = x2d.shape
    dout = W.shape[1]
    return pl.pallas_call(
        _proj_kernel,
        grid=(n // blk,),
        in_specs=[
            pl.BlockSpec((blk, d), lambda i: (i, 0)),
            pl.BlockSpec((d, dout), lambda i: (0, 0)),
            pl.BlockSpec((dout,), lambda i: (0,)),
        ],
        out_specs=pl.BlockSpec((blk, dout), lambda i: (i, 0)),
        out_shape=jax.ShapeDtypeStruct((n, dout), jnp.float32),
    )(x2d, W, b)


_NCH = _NQ // _QC
_CW = _NK * _QC
_OW = _QC * 16
_VTW = _NV * _PITCH
_KU = 1


def _sc_gather_body(vt_hbm, idx_hbm, w_hbm, out_hbm, vt_v, idx_v, w_v, out_v):
    cid = lax.axis_index("c")
    sid = lax.axis_index("s")
    wid = sid * 2 + cid
    b = wid // 16
    h = (wid // 2) % 8
    pltpu.sync_copy(vt_hbm.at[pl.ds(wid * _VTW, _VTW)], vt_v)

    def chunk_body(ci, _):
        hoff = pl.multiple_of(h * 64, 64)
        pltpu.sync_copy(idx_hbm.at[b, ci, pl.ds(hoff, _NK), :], idx_v)
        pltpu.sync_copy(w_hbm.at[b, ci, pl.ds(hoff, _NK), :], w_v)

        def group_body(g, _):
            ql = g * 16
            def k_body(ko, accs):
                for ku in range(_KU):
                    k = ko * _KU + ku
                    idxv = idx_v[k, pl.ds(ql, 16)].astype(jnp.int32)
                    wvec = w_v[k, pl.ds(ql, 16)]
                    accs = tuple(
                        accs[c] + wvec * plsc.load_gather(vt_v, [idxv + c])
                        for c in range(16))
                return accs
            accs = lax.fori_loop(
                0, _NK // _KU, k_body,
                tuple(jnp.zeros((16,), jnp.float32) for _ in range(16)))
            rows = (lax.iota(jnp.int32, 16) + ql) * 16
            for c in range(16):
                plsc.store_scatter(out_v, [rows + c], accs[c])
            return 0

        lax.fori_loop(0, _QC // 16, group_body, 0)
        pltpu.sync_copy(out_v, out_hbm.at[pl.ds((wid * _NCH + ci) * _OW, _OW)])
        return 0

    lax.fori_loop(0, _NCH, chunk_body, 0)


@functools.cache
def _sc_gather_build():
    return pl.kernel(
        _sc_gather_body,
        out_type=jax.ShapeDtypeStruct((_BS * _NH * 2 * _NQ * 16,), jnp.float32),
        mesh=plsc.VectorSubcoreMesh(core_axis_name="c", subcore_axis_name="s",
                                    num_cores=2, num_subcores=16),
        compiler_params=pltpu.CompilerParams(needs_layout_passes=False),
        scratch_types=[
            pltpu.VMEM((_VTW,), jnp.float32),
            pltpu.VMEM((_NK, _QC), jnp.float32),
            pltpu.VMEM((_NK, _QC), jnp.float32),
            pltpu.VMEM((_OW,), jnp.float32),
        ],
    )


def _sc_gather(vt, idx_sc, w_sc):
    return _sc_gather_build()(vt, idx_sc, w_sc)


def kernel(query, value, reference_points, W_value, b_value, W_off, b_off,
           W_attn, b_attn, W_out, b_out):
    woffp = W_off[:, _PERM]
    boffp = b_off[_PERM]
    woffx, woffy = woffp[:, :128], woffp[:, 128:]
    boffx, boffy = boffp[:128], boffp[128:]
    rp = jnp.broadcast_to(reference_points[:, :, None, :, None, :],
                          (_BS, _NQ, _NH, _NL, _NP, 2))
    rpx = rp[..., 0].reshape(_BS, _NQ, 128)
    rpy = rp[..., 1].reshape(_BS, _NQ, 128)

    idx_sc, w_sc = _prep(
        query, rpx, rpy, woffx, woffy, boffx, boffy, W_attn, b_attn)

    v = _proj(value.reshape(_BS * _NV, _D), W_value, b_value)
    vt = jnp.transpose(v.reshape(_BS, _NV, _NH, 2, 16),
                       (0, 2, 3, 1, 4))
    vt = jnp.pad(vt, ((0, 0), (0, 0), (0, 0), (0, 0),
                      (0, _PITCH - 16))).reshape(-1)

    out_sc = _sc_gather(vt, idx_sc, w_sc)

    out2d = jnp.transpose(out_sc.reshape(_BS, _NH, 2, _NCH, _QC, 16),
                          (0, 3, 4, 1, 2, 5)).reshape(_BS * _NQ, _D)
    return _proj(out2d, W_out, b_out).reshape(_BS, _NQ, _D)


# speedup vs baseline: 2.2921x; 1.0910x over previous
import functools

import jax
import jax.numpy as jnp
import numpy as np
from jax import lax
from jax.experimental import pallas as pl
from jax.experimental.pallas import tpu as pltpu
from jax.experimental.pallas import tpu_sc as plsc

_BS = 2
_NQ = 5440
_NV = 5440
_D = 256
_NH = 8
_NL = 4
_NP = 4
_HD = 32
_SPATIAL = [(64, 64), (32, 32), (16, 16), (8, 8)]
_SIZES = [h * w for (h, w) in _SPATIAL]
_STARTS = np.concatenate([[0], np.cumsum(_SIZES)]).astype(np.int32)
_NK = _NL * _NP * 4
_QB = 680
_QC = 160

_lidx = np.tile(np.repeat(np.arange(_NL), _NP), _NH)
_Wc = np.array([_SPATIAL[l][1] for l in _lidx], np.float32)
_Hc = np.array([_SPATIAL[l][0] for l in _lidx], np.float32)
_S16c = np.array([_STARTS[l] for l in _lidx], np.float32)

_cols = np.arange(_NH * _NL * _NP * 2).reshape(_NH, _NL, _NP, 2)
_PERM = np.concatenate([_cols[..., 0].reshape(-1), _cols[..., 1].reshape(-1)])

_BD = np.kron(np.eye(_NH, dtype=np.float32), np.ones((16, 16), np.float32))

_CONSTS = np.zeros((8, 128), np.float32)
_CONSTS[0] = _Wc
_CONSTS[1] = _Hc
_CONSTS[2] = _S16c


def _prep_kernel(q_ref, rpx_ref, rpy_ref, woffx_ref, woffy_ref, boffx_ref,
                 boffy_ref, wattn_ref, battn_ref, bd_ref, consts_ref,
                 idx_ref, w_ref):
    q = q_ref[0]
    offx = jnp.dot(q, woffx_ref[...], preferred_element_type=jnp.float32) + boffx_ref[...]
    offy = jnp.dot(q, woffy_ref[...], preferred_element_type=jnp.float32) + boffy_ref[...]
    awr = jnp.dot(q, wattn_ref[...], preferred_element_type=jnp.float32) + battn_ref[...]
    e = jnp.exp(awr)
    s = jnp.dot(e, bd_ref[...], preferred_element_type=jnp.float32)
    aw = e / s

    wv = consts_ref[0:1, :]
    hv = consts_ref[1:2, :]
    s16 = consts_ref[2:3, :]
    gx = 2.0 * (rpx_ref[0] + offx * (1.0 / wv)) - 1.0
    gy = 2.0 * (rpy_ref[0] + offy * (1.0 / hv)) - 1.0
    x = ((gx + 1.0) * wv - 1.0) * 0.5
    y = ((gy + 1.0) * hv - 1.0) * 0.5

    x0 = jnp.floor(x)
    y0 = jnp.floor(y)
    lx = x - x0
    ly = y - y0
    hx = 1.0 - lx
    hy = 1.0 - ly

    corners = ((x0, y0, hx * hy),
               (x0 + 1.0, y0, lx * hy),
               (x0, y0 + 1.0, hx * ly),
               (x0 + 1.0, y0 + 1.0, lx * ly))
    idx_t, w_t = [], []
    for xi, yi, bw in corners:
        valid = ((xi >= 0.0) & (xi <= wv - 1.0)
                 & (yi >= 0.0) & (yi <= hv - 1.0)).astype(jnp.float32)
        xc = jnp.clip(xi, 0.0, wv - 1.0)
        yc = jnp.clip(yi, 0.0, hv - 1.0)
        idx16 = (yc * wv + xc) + s16
        idx_t.append(jnp.transpose(idx16, (1, 0)))
        w_t.append(jnp.transpose(bw * valid * aw, (1, 0)))
    idx_rows = jnp.concatenate(
        [t[h * 16:(h + 1) * 16] for h in range(_NH) for t in idx_t], axis=0)
    w_rows = jnp.concatenate(
        [t[h * 16:(h + 1) * 16] for h in range(_NH) for t in w_t], axis=0)
    idx_ref[0, 0] = idx_rows
    w_ref[0, 0] = w_rows


def _prep(query, rpx, rpy, woffx, woffy, boffx, boffy, wattn, battn):
    sc_arr = jax.ShapeDtypeStruct((_BS, _NQ // _QC, 512, _QC), jnp.float32)
    blk = lambda: pl.BlockSpec((1, _QC, 128), lambda b, i: (b, i, 0))
    return pl.pallas_call(
        _prep_kernel,
        grid=(_BS, _NQ // _QC),
        in_specs=[
            pl.BlockSpec((1, _QC, _D), lambda b, i: (b, i, 0)),
            blk(), blk(),
            pl.BlockSpec((_D, 128), lambda b, i: (0, 0)),
            pl.BlockSpec((_D, 128), lambda b, i: (0, 0)),
            pl.BlockSpec((128,), lambda b, i: (0,)),
            pl.BlockSpec((128,), lambda b, i: (0,)),
            pl.BlockSpec((_D, 128), lambda b, i: (0, 0)),
            pl.BlockSpec((128,), lambda b, i: (0,)),
            pl.BlockSpec((128, 128), lambda b, i: (0, 0)),
            pl.BlockSpec((8, 128), lambda b, i: (0, 0)),
        ],
        out_specs=[pl.BlockSpec((1, 1, 512, _QC), lambda b, i: (b, i, 0, 0))] * 2,
        out_shape=[sc_arr, sc_arr],
    )(query, rpx, rpy, woffx, woffy, boffx, boffy, wattn, battn,
      jnp.asarray(_BD), jnp.asarray(_CONSTS))


def _vproj_kernel(x_ref, w_ref, b_ref, o_ref):
    y = lax.dot_general(w_ref[...], x_ref[0], (((0,), (1,)), ((), ())),
                        preferred_element_type=jnp.float32)
    o_ref[0] = y + b_ref[...][:, None]


def _vproj(value, W, b):
    return pl.pallas_call(
        _vproj_kernel,
        grid=(_BS,),
        in_specs=[
            pl.BlockSpec((1, _NV, _D), lambda i: (i, 0, 0)),
            pl.BlockSpec((_D, _D), lambda i: (0, 0)),
            pl.BlockSpec((_D,), lambda i: (0,)),
        ],
        out_specs=pl.BlockSpec((1, _D, _NV), lambda i: (i, 0, 0)),
        out_shape=jax.ShapeDtypeStruct((_BS, _D, _NV), jnp.float32),
    )(value, W, b)


def _oproj_kernel(x_ref, w_ref, b_ref, o_ref):
    xt = x_ref[0, :, 0].reshape(_D, _QC)
    y = lax.dot_general(xt, w_ref[...], (((0,), (0,)), ((), ())),
                        preferred_element_type=jnp.float32)
    o_ref[0] = y + b_ref[...]


def _oproj(x5, W, b):
    return pl.pallas_call(
        _oproj_kernel,
        grid=(_BS, _NQ // _QC),
        in_specs=[
            pl.BlockSpec((1, 16, 1, 16, _QC), lambda bq, i: (bq, 0, i, 0, 0)),
            pl.BlockSpec((_D, _D), lambda bq, i: (0, 0)),
            pl.BlockSpec((_D,), lambda bq, i: (0,)),
        ],
        out_specs=pl.BlockSpec((1, _QC, _D), lambda bq, i: (bq, i, 0)),
        out_shape=jax.ShapeDtypeStruct((_BS, _NQ, _D), jnp.float32),
    )(x5, W, b)


_NCH = _NQ // _QC
_CW = _NK * _QC
_OW = _QC * 16
_VTW = _NV * 16
_KU = 1


def _sc_gather_body(vt_hbm, idx_hbm, w_hbm, out_hbm, vt_v, idx_v, w_v, out_v):
    cid = lax.axis_index("c")
    sid = lax.axis_index("s")
    wid = sid * 2 + cid
    b = wid // 16
    h = (wid // 2) % 8
    pltpu.sync_copy(vt_hbm.at[pl.ds(wid * _VTW, _VTW)], vt_v)

    def chunk_body(ci, _):
        hoff = pl.multiple_of(h * 64, 64)
        pltpu.sync_copy(idx_hbm.at[b, ci, pl.ds(hoff, _NK), :], idx_v)
        pltpu.sync_copy(w_hbm.at[b, ci, pl.ds(hoff, _NK), :], w_v)

        def group_body(g, _):
            ql = g * 16
            def k_body(ko, accs):
                for ku in range(_KU):
                    k = ko * _KU + ku
                    idxv = idx_v[k, pl.ds(ql, 16)].astype(jnp.int32)
                    wvec = w_v[k, pl.ds(ql, 16)]
                    accs = tuple(
                        accs[c] + wvec * plsc.load_gather(
                            vt_v.at[pl.ds(c * _NV, _NV)], [idxv])
                        for c in range(16))
                return accs
            accs = lax.fori_loop(
                0, _NK // _KU, k_body,
                tuple(jnp.zeros((16,), jnp.float32) for _ in range(16)))
            for c in range(16):
                out_v[pl.ds(c * _QC + ql, 16)] = accs[c]
            return 0

        lax.fori_loop(0, _QC // 16, group_body, 0)
        pltpu.sync_copy(out_v, out_hbm.at[pl.ds((wid * _NCH + ci) * _OW, _OW)])
        return 0

    lax.fori_loop(0, _NCH, chunk_body, 0)


@functools.cache
def _sc_gather_build():
    return pl.kernel(
        _sc_gather_body,
        out_type=jax.ShapeDtypeStruct((_BS * _NH * 2 * _NQ * 16,), jnp.float32),
        mesh=plsc.VectorSubcoreMesh(core_axis_name="c", subcore_axis_name="s",
                                    num_cores=2, num_subcores=16),
        compiler_params=pltpu.CompilerParams(needs_layout_passes=False),
        scratch_types=[
            pltpu.VMEM((_VTW,), jnp.float32),
            pltpu.VMEM((_NK, _QC), jnp.float32),
            pltpu.VMEM((_NK, _QC), jnp.float32),
            pltpu.VMEM((_OW,), jnp.float32),
        ],
    )


def _sc_gather(vt, idx_sc, w_sc):
    return _sc_gather_build()(vt, idx_sc, w_sc)


def kernel(query, value, reference_points, W_value, b_value, W_off, b_off,
           W_attn, b_attn, W_out, b_out):
    woffp = W_off[:, _PERM]
    boffp = b_off[_PERM]
    woffx, woffy = woffp[:, :128], woffp[:, 128:]
    boffx, boffy = boffp[:128], boffp[128:]
    rp = jnp.broadcast_to(reference_points[:, :, None, :, None, :],
                          (_BS, _NQ, _NH, _NL, _NP, 2))
    rpx = rp[..., 0].reshape(_BS, _NQ, 128)
    rpy = rp[..., 1].reshape(_BS, _NQ, 128)

    idx_sc, w_sc = _prep(
        query, rpx, rpy, woffx, woffy, boffx, boffy, W_attn, b_attn)

    vt = _vproj(value, W_value, b_value).reshape(-1)

    out_sc = _sc_gather(vt, idx_sc, w_sc)

    x5 = out_sc.reshape(_BS, 16, _NCH, 16, _QC)
    return _oproj(x5, W_out, b_out)


# speedup vs baseline: 2.2995x; 1.0032x over previous
import functools

import jax
import jax.numpy as jnp
import numpy as np
from jax import lax
from jax.experimental import pallas as pl
from jax.experimental.pallas import tpu as pltpu
from jax.experimental.pallas import tpu_sc as plsc

_BS = 2
_NQ = 5440
_NV = 5440
_D = 256
_NH = 8
_NL = 4
_NP = 4
_HD = 32
_SPATIAL = [(64, 64), (32, 32), (16, 16), (8, 8)]
_SIZES = [h * w for (h, w) in _SPATIAL]
_STARTS = np.concatenate([[0], np.cumsum(_SIZES)]).astype(np.int32)
_NK = _NL * _NP * 4
_QB = 680
_QC = 160

_lidx = np.tile(np.repeat(np.arange(_NL), _NP), _NH)
_Wc = np.array([_SPATIAL[l][1] for l in _lidx], np.float32)
_Hc = np.array([_SPATIAL[l][0] for l in _lidx], np.float32)
_S16c = np.array([_STARTS[l] for l in _lidx], np.float32)

_cols = np.arange(_NH * _NL * _NP * 2).reshape(_NH, _NL, _NP, 2)
_PERM = np.concatenate([_cols[..., 0].reshape(-1), _cols[..., 1].reshape(-1)])

_BD = np.kron(np.eye(_NH, dtype=np.float32), np.ones((16, 16), np.float32))

_CONSTS = np.zeros((8, 128), np.float32)
_CONSTS[0] = _Wc
_CONSTS[1] = _Hc
_CONSTS[2] = _S16c


def _prep_kernel(q_ref, rpx_ref, rpy_ref, woffx_ref, woffy_ref, boffx_ref,
                 boffy_ref, wattn_ref, battn_ref, bd_ref, consts_ref,
                 idx_ref, w_ref):
    q = q_ref[0]
    offx = jnp.dot(q, woffx_ref[...], preferred_element_type=jnp.float32) + boffx_ref[...]
    offy = jnp.dot(q, woffy_ref[...], preferred_element_type=jnp.float32) + boffy_ref[...]
    awr = jnp.dot(q, wattn_ref[...], preferred_element_type=jnp.float32) + battn_ref[...]
    e = jnp.exp(awr)
    s = jnp.dot(e, bd_ref[...], preferred_element_type=jnp.float32)
    aw = e / s

    wv = consts_ref[0:1, :]
    hv = consts_ref[1:2, :]
    s16 = consts_ref[2:3, :]
    gx = 2.0 * (rpx_ref[0] + offx * (1.0 / wv)) - 1.0
    gy = 2.0 * (rpy_ref[0] + offy * (1.0 / hv)) - 1.0
    x = ((gx + 1.0) * wv - 1.0) * 0.5
    y = ((gy + 1.0) * hv - 1.0) * 0.5

    x0 = jnp.floor(x)
    y0 = jnp.floor(y)
    lx = x - x0
    ly = y - y0
    hx = 1.0 - lx
    hy = 1.0 - ly

    corners = ((x0, y0, hx * hy),
               (x0 + 1.0, y0, lx * hy),
               (x0, y0 + 1.0, hx * ly),
               (x0 + 1.0, y0 + 1.0, lx * ly))
    idx_t, w_t = [], []
    for xi, yi, bw in corners:
        valid = ((xi >= 0.0) & (xi <= wv - 1.0)
                 & (yi >= 0.0) & (yi <= hv - 1.0)).astype(jnp.float32)
        xc = jnp.clip(xi, 0.0, wv - 1.0)
        yc = jnp.clip(yi, 0.0, hv - 1.0)
        idx16 = (yc * wv + xc) + s16
        idx_t.append(jnp.transpose(idx16, (1, 0)))
        w_t.append(jnp.transpose(bw * valid * aw, (1, 0)))
    idx_rows = jnp.concatenate(
        [t[h * 16:(h + 1) * 16] for h in range(_NH) for t in idx_t], axis=0)
    w_rows = jnp.concatenate(
        [t[h * 16:(h + 1) * 16] for h in range(_NH) for t in w_t], axis=0)
    idx_ref[0, 0] = idx_rows
    w_ref[0, 0] = w_rows


def _prep(query, rpx, rpy, woffx, woffy, boffx, boffy, wattn, battn):
    sc_arr = jax.ShapeDtypeStruct((_BS, _NQ // _QC, 512, _QC), jnp.float32)
    blk = lambda: pl.BlockSpec((1, _QC, 128), lambda b, i: (b, i, 0))
    return pl.pallas_call(
        _prep_kernel,
        grid=(_BS, _NQ // _QC),
        in_specs=[
            pl.BlockSpec((1, _QC, _D), lambda b, i: (b, i, 0)),
            blk(), blk(),
            pl.BlockSpec((_D, 128), lambda b, i: (0, 0)),
            pl.BlockSpec((_D, 128), lambda b, i: (0, 0)),
            pl.BlockSpec((128,), lambda b, i: (0,)),
            pl.BlockSpec((128,), lambda b, i: (0,)),
            pl.BlockSpec((_D, 128), lambda b, i: (0, 0)),
            pl.BlockSpec((128,), lambda b, i: (0,)),
            pl.BlockSpec((128, 128), lambda b, i: (0, 0)),
            pl.BlockSpec((8, 128), lambda b, i: (0, 0)),
        ],
        out_specs=[pl.BlockSpec((1, 1, 512, _QC), lambda b, i: (b, i, 0, 0))] * 2,
        out_shape=[sc_arr, sc_arr],
    )(query, rpx, rpy, woffx, woffy, boffx, boffy, wattn, battn,
      jnp.asarray(_BD), jnp.asarray(_CONSTS))


def _vproj_kernel(x_ref, w_ref, b_ref, o_ref):
    y = lax.dot_general(w_ref[...], x_ref[0], (((0,), (1,)), ((), ())),
                        preferred_element_type=jnp.float32)
    o_ref[0] = y + b_ref[...][:, None]


def _vproj(value, W, b):
    return pl.pallas_call(
        _vproj_kernel,
        grid=(_BS,),
        in_specs=[
            pl.BlockSpec((1, _NV, _D), lambda i: (i, 0, 0)),
            pl.BlockSpec((_D, _D), lambda i: (0, 0)),
            pl.BlockSpec((_D,), lambda i: (0,)),
        ],
        out_specs=pl.BlockSpec((1, _D, _NV), lambda i: (i, 0, 0)),
        out_shape=jax.ShapeDtypeStruct((_BS, _D, _NV), jnp.float32),
    )(value, W, b)


def _oproj_kernel(x_ref, w_ref, b_ref, o_ref):
    xt = x_ref[0, :, 0].reshape(_D, _QC)
    y = lax.dot_general(xt, w_ref[...], (((0,), (0,)), ((), ())),
                        preferred_element_type=jnp.float32)
    o_ref[0] = y + b_ref[...]


def _oproj(x5, W, b):
    return pl.pallas_call(
        _oproj_kernel,
        grid=(_BS, _NQ // _QC),
        in_specs=[
            pl.BlockSpec((1, 16, 1, 16, _QC), lambda bq, i: (bq, 0, i, 0, 0)),
            pl.BlockSpec((_D, _D), lambda bq, i: (0, 0)),
            pl.BlockSpec((_D,), lambda bq, i: (0,)),
        ],
        out_specs=pl.BlockSpec((1, _QC, _D), lambda bq, i: (bq, i, 0)),
        out_shape=jax.ShapeDtypeStruct((_BS, _NQ, _D), jnp.float32),
    )(x5, W, b)


_NCH = _NQ // _QC
_CW = _NK * _QC
_OW = _QC * 16
_VTW = _NV * 16
_KU = 1


def _sc_gather_body(vt_hbm, idx_hbm, w_hbm, out_hbm, vt_v, idx_v, w_v, out_v):
    cid = lax.axis_index("c")
    sid = lax.axis_index("s")
    wid = sid * 2 + cid
    b = wid // 16
    h = (wid // 2) % 8
    pltpu.sync_copy(vt_hbm.at[pl.ds(wid * _VTW, _VTW)], vt_v)

    def chunk_body(ci, _):
        hoff = pl.multiple_of(h * 64, 64)
        pltpu.sync_copy(idx_hbm.at[b, ci, pl.ds(hoff, _NK), :], idx_v)
        pltpu.sync_copy(w_hbm.at[b, ci, pl.ds(hoff, _NK), :], w_v)

        def group_body(g, _):
            ql = g * 16
            def k_body(ko, accs):
                for ku in range(_KU):
                    k = ko * _KU + ku
                    idxv = idx_v[k, pl.ds(ql, 16)].astype(jnp.int32)
                    wvec = w_v[k, pl.ds(ql, 16)]
                    accs = tuple(
                        accs[c] + wvec * plsc.load_gather(vt_v, [idxv + c * _NV])
                        for c in range(16))
                return accs
            accs = lax.fori_loop(
                0, _NK // _KU, k_body,
                tuple(jnp.zeros((16,), jnp.float32) for _ in range(16)))
            for c in range(16):
                out_v[pl.ds(c * _QC + ql, 16)] = accs[c]
            return 0

        lax.fori_loop(0, _QC // 16, group_body, 0)
        pltpu.sync_copy(out_v, out_hbm.at[pl.ds((wid * _NCH + ci) * _OW, _OW)])
        return 0

    lax.fori_loop(0, _NCH, chunk_body, 0)


@functools.cache
def _sc_gather_build():
    return pl.kernel(
        _sc_gather_body,
        out_type=jax.ShapeDtypeStruct((_BS * _NH * 2 * _NQ * 16,), jnp.float32),
        mesh=plsc.VectorSubcoreMesh(core_axis_name="c", subcore_axis_name="s",
                                    num_cores=2, num_subcores=16),
        compiler_params=pltpu.CompilerParams(needs_layout_passes=False),
        scratch_types=[
            pltpu.VMEM((_VTW,), jnp.float32),
            pltpu.VMEM((_NK, _QC), jnp.float32),
            pltpu.VMEM((_NK, _QC), jnp.float32),
            pltpu.VMEM((_OW,), jnp.float32),
        ],
    )


def _sc_gather(vt, idx_sc, w_sc):
    return _sc_gather_build()(vt, idx_sc, w_sc)


def kernel(query, value, reference_points, W_value, b_value, W_off, b_off,
           W_attn, b_attn, W_out, b_out):
    woffp = W_off[:, _PERM]
    boffp = b_off[_PERM]
    woffx, woffy = woffp[:, :128], woffp[:, 128:]
    boffx, boffy = boffp[:128], boffp[128:]
    rp = jnp.broadcast_to(reference_points[:, :, None, :, None, :],
                          (_BS, _NQ, _NH, _NL, _NP, 2))
    rpx = rp[..., 0].reshape(_BS, _NQ, 128)
    rpy = rp[..., 1].reshape(_BS, _NQ, 128)

    idx_sc, w_sc = _prep(
        query, rpx, rpy, woffx, woffy, boffx, boffy, W_attn, b_attn)

    vt = _vproj(value, W_value, b_value).reshape(-1)

    out_sc = _sc_gather(vt, idx_sc, w_sc)

    x5 = out_sc.reshape(_BS, 16, _NCH, 16, _QC)
    return _oproj(x5, W_out, b_out)
